# Initial kernel scaffold; baseline (speedup 1.0000x reference)
import jax
import jax.numpy as jnp
from jax.experimental import pallas as pl


def kernel(u):
    raise NotImplementedError("write your pallas kernel here")


# speedup vs baseline: 1.0126x; 1.0126x over previous
import jax
import jax.numpy as jnp
from jax.experimental import pallas as pl

_K = 0


def kernel(u):
    B, S, D = u.shape

    def body(u_ref, o_ref):
        o_ref[:, 0, :] = u_ref[:, _K % 8, :]

    return pl.pallas_call(
        body,
        grid=(1,),
        in_specs=[pl.BlockSpec((B, 8, D), lambda i: (0, _K // 8, 0))],
        out_specs=pl.BlockSpec((B, 1, D), lambda i: (0, 0, 0)),
        out_shape=jax.ShapeDtypeStruct((B, 1, D), u.dtype),
    )(u)
